# baseline (speedup 1.0000x reference)
import functools

import jax
import jax.numpy as jnp
import numpy as np
from jax.experimental import pallas as pl
from jax.experimental.pallas import tpu as pltpu

_BLOCK = 32
_NLOCAL = 2
_STRIDE = 8
_TQ = 256
_SUPER = _STRIDE * _BLOCK
_LOCW = _TQ + 2 * _BLOCK
_NEG = -1e30


def _local_start(t, S):
    return min(max(t * _TQ - _BLOCK, 0), S - _LOCW)


def _make_biases(T, S):
    ns = (S // _SUPER) * _BLOCK
    rows = np.arange(T)[:, None] // _BLOCK
    cs = np.arange(ns)[None, :] // _BLOCK * _STRIDE
    bias_s = np.where(np.abs(rows - cs) >= _NLOCAL, 0.0, _NEG).astype(np.float32)

    bias_l = np.full((T, _LOCW), _NEG, dtype=np.float32)
    for t in range(T // _TQ):
        start = _local_start(t, S)
        r = np.arange(t * _TQ, (t + 1) * _TQ)[:, None] // _BLOCK
        c = start // _BLOCK + np.arange(_LOCW)[None, :] // _BLOCK
        bias_l[t * _TQ:(t + 1) * _TQ] = np.where(
            np.abs(r - c) < _NLOCAL, 0.0, _NEG)
    return bias_s, bias_l


def _attn_kernel(H, E, q_ref, k_ref, v_ref, bs_ref, bl_ref, o_ref,
                 ks_ref, vs_ref):
    t = pl.program_id(0)
    S = k_ref.shape[0]
    n_super = S // _SUPER
    temp = 1.0 / float(np.sqrt(E))

    @pl.when(t == 0)
    def _gather():
        for i in range(n_super):
            ks_ref[:, i * _BLOCK:(i + 1) * _BLOCK, :] = \
                jnp.transpose(k_ref[i * _SUPER:i * _SUPER + _BLOCK], (1, 0, 2))
            vs_ref[:, i * _BLOCK:(i + 1) * _BLOCK, :] = \
                jnp.transpose(v_ref[i * _SUPER:i * _SUPER + _BLOCK], (1, 0, 2))

    start = pl.multiple_of(jnp.clip(t * _TQ - _BLOCK, 0, S - _LOCW), _BLOCK)
    bs = bs_ref[...]
    bl = bl_ref[...]

    qt = jnp.transpose(q_ref[...], (1, 0, 2)) * temp
    klt = jnp.transpose(k_ref[pl.ds(start, _LOCW)], (1, 0, 2))
    vlt = jnp.transpose(v_ref[pl.ds(start, _LOCW)], (1, 0, 2))

    dn = (((1,), (1,)), ((), ()))
    dv = (((1,), (0,)), ((), ()))
    outs = []
    for h in range(H):
        q = qt[h]
        ks = ks_ref[h]
        vs = vs_ref[h]
        kl = klt[h]
        vl = vlt[h]

        ss = jax.lax.dot_general(q, ks, dn,
                                 preferred_element_type=jnp.float32) + bs
        sl = jax.lax.dot_general(q, kl, dn,
                                 preferred_element_type=jnp.float32) + bl

        m = jnp.maximum(jnp.max(ss, axis=1), jnp.max(sl, axis=1))
        ps = jnp.exp(ss - m[:, None])
        plc = jnp.exp(sl - m[:, None])
        denom = jnp.sum(ps, axis=1) + jnp.sum(plc, axis=1)

        out = jax.lax.dot_general(ps, vs, dv,
                                  preferred_element_type=jnp.float32)
        out = out + jax.lax.dot_general(plc, vl, dv,
                                        preferred_element_type=jnp.float32)
        outs.append(out / denom[:, None])
    o_ref[...] = jnp.transpose(jnp.stack(outs, axis=0), (1, 0, 2))


def kernel(query, key, value):
    B, T, H, E = query.shape
    S = key.shape[1]
    q = query[0]
    k = key[0]
    v = value[0]
    ns = (S // _SUPER) * _BLOCK
    bias_s, bias_l = _make_biases(T, S)

    out = pl.pallas_call(
        functools.partial(_attn_kernel, H, E),
        grid=(T // _TQ,),
        in_specs=[
            pl.BlockSpec((_TQ, H, E), lambda t: (t, 0, 0)),
            pl.BlockSpec((S, H, E), lambda t: (0, 0, 0)),
            pl.BlockSpec((S, H, E), lambda t: (0, 0, 0)),
            pl.BlockSpec((_TQ, ns), lambda t: (t, 0)),
            pl.BlockSpec((_TQ, _LOCW), lambda t: (t, 0)),
        ],
        out_specs=pl.BlockSpec((_TQ, H, E), lambda t: (t, 0, 0)),
        out_shape=jax.ShapeDtypeStruct((T, H, E), jnp.float32),
        scratch_shapes=[
            pltpu.VMEM((H, ns, E), jnp.float32),
            pltpu.VMEM((H, ns, E), jnp.float32),
        ],
    )(q, k, v, jnp.asarray(bias_s), jnp.asarray(bias_l))
    return out[None]


# speedup vs baseline: 1.0149x; 1.0149x over previous
import functools

import jax
import jax.numpy as jnp
import numpy as np
from jax.experimental import pallas as pl
from jax.experimental.pallas import tpu as pltpu

_BLOCK = 32
_NLOCAL = 2
_STRIDE = 8
_TQ = 256
_SUPER = _STRIDE * _BLOCK
_LOCW = _TQ + 2 * _BLOCK
_NEG = -1e30


def _local_start(t, S):
    return min(max(t * _TQ - _BLOCK, 0), S - _LOCW)


def _make_biases(T, S):
    ns = (S // _SUPER) * _BLOCK
    rows = np.arange(T)[:, None] // _BLOCK
    cs = np.arange(ns)[None, :] // _BLOCK * _STRIDE
    bias_s = np.where(np.abs(rows - cs) >= _NLOCAL, 0.0, _NEG).astype(np.float32)

    bias_l = np.full((T, _LOCW), _NEG, dtype=np.float32)
    for t in range(T // _TQ):
        start = _local_start(t, S)
        r = np.arange(t * _TQ, (t + 1) * _TQ)[:, None] // _BLOCK
        c = start // _BLOCK + np.arange(_LOCW)[None, :] // _BLOCK
        bias_l[t * _TQ:(t + 1) * _TQ] = np.where(
            np.abs(r - c) < _NLOCAL, 0.0, _NEG)
    return bias_s, bias_l


def _attn_kernel(H, E, q_ref, k_ref, v_ref, bs_ref, bl_ref, o_ref,
                 ks_ref, vs_ref):
    t = pl.program_id(0)
    S = k_ref.shape[1]
    n_super = S // _SUPER
    temp = 1.0 / float(np.sqrt(E))

    @pl.when(t == 0)
    def _gather():
        for i in range(n_super):
            ks_ref[:, i * _BLOCK:(i + 1) * _BLOCK, :] = \
                jnp.transpose(k_ref[0, i * _SUPER:i * _SUPER + _BLOCK], (1, 0, 2))
            vs_ref[:, i * _BLOCK:(i + 1) * _BLOCK, :] = \
                jnp.transpose(v_ref[0, i * _SUPER:i * _SUPER + _BLOCK], (1, 0, 2))

    start = pl.multiple_of(jnp.clip(t * _TQ - _BLOCK, 0, S - _LOCW), _BLOCK)
    bs = bs_ref[...]
    bl = bl_ref[...]

    qt = jnp.transpose(q_ref[0], (1, 0, 2)) * temp
    klt = jnp.transpose(k_ref[0, pl.ds(start, _LOCW)], (1, 0, 2))
    vlt = jnp.transpose(v_ref[0, pl.ds(start, _LOCW)], (1, 0, 2))

    dn = (((1,), (1,)), ((), ()))
    dv = (((1,), (0,)), ((), ()))
    outs = []
    for h in range(H):
        q = qt[h]
        ks = ks_ref[h]
        vs = vs_ref[h]
        kl = klt[h]
        vl = vlt[h]

        ss = jax.lax.dot_general(q, ks, dn,
                                 preferred_element_type=jnp.float32) + bs
        sl = jax.lax.dot_general(q, kl, dn,
                                 preferred_element_type=jnp.float32) + bl

        m = jnp.maximum(jnp.max(ss, axis=1), jnp.max(sl, axis=1))
        ps = jnp.exp(ss - m[:, None])
        plc = jnp.exp(sl - m[:, None])
        denom = jnp.sum(ps, axis=1) + jnp.sum(plc, axis=1)

        out = jax.lax.dot_general(ps, vs, dv,
                                  preferred_element_type=jnp.float32)
        out = out + jax.lax.dot_general(plc, vl, dv,
                                        preferred_element_type=jnp.float32)
        outs.append(out / denom[:, None])
    o_ref[0] = jnp.transpose(jnp.stack(outs, axis=0), (1, 0, 2))


def kernel(query, key, value):
    B, T, H, E = query.shape
    S = key.shape[1]
    ns = (S // _SUPER) * _BLOCK
    bias_s, bias_l = _make_biases(T, S)

    out = pl.pallas_call(
        functools.partial(_attn_kernel, H, E),
        grid=(T // _TQ,),
        in_specs=[
            pl.BlockSpec((1, _TQ, H, E), lambda t: (0, t, 0, 0)),
            pl.BlockSpec((1, S, H, E), lambda t: (0, 0, 0, 0)),
            pl.BlockSpec((1, S, H, E), lambda t: (0, 0, 0, 0)),
            pl.BlockSpec((_TQ, ns), lambda t: (t, 0)),
            pl.BlockSpec((_TQ, _LOCW), lambda t: (t, 0)),
        ],
        out_specs=pl.BlockSpec((1, _TQ, H, E), lambda t: (0, t, 0, 0)),
        out_shape=jax.ShapeDtypeStruct((1, T, H, E), jnp.float32),
        scratch_shapes=[
            pltpu.VMEM((H, ns, E), jnp.float32),
            pltpu.VMEM((H, ns, E), jnp.float32),
        ],
    )(query, key, value, jnp.asarray(bias_s), jnp.asarray(bias_l))
    return out


# speedup vs baseline: 1.1495x; 1.1327x over previous
import functools

import jax
import jax.numpy as jnp
import numpy as np
from jax.experimental import pallas as pl
from jax.experimental.pallas import tpu as pltpu

_BLOCK = 32
_NLOCAL = 2
_STRIDE = 8
_TQ = 256
_SUPER = _STRIDE * _BLOCK
_LOCW = 448
_ALIGN = 128
_NEG = -1e30


def _local_start(t, S):
    return min(max(t * _TQ - _ALIGN, 0), S - _LOCW)


def _make_biases(T, S):
    ns = (S // _SUPER) * _BLOCK
    rows = np.arange(T)[None, :] // _BLOCK
    cs = np.arange(ns)[:, None] // _BLOCK * _STRIDE
    bias_s = np.where(np.abs(rows - cs) >= _NLOCAL, 0.0, _NEG).astype(np.float32)

    bias_l = np.full((_LOCW, T), _NEG, dtype=np.float32)
    for t in range(T // _TQ):
        start = _local_start(t, S)
        r = np.arange(t * _TQ, (t + 1) * _TQ)[None, :] // _BLOCK
        c = start // _BLOCK + np.arange(_LOCW)[:, None] // _BLOCK
        bias_l[:, t * _TQ:(t + 1) * _TQ] = np.where(
            np.abs(r - c) < _NLOCAL, 0.0, _NEG)
    return bias_s, bias_l


def _attn_kernel(H, E, q_ref, k_ref, v_ref, bs_ref, bl_ref, o_ref,
                 ks_ref, vs_ref):
    t = pl.program_id(1)
    S = k_ref.shape[2]
    n_super = S // _SUPER
    temp = 1.0 / float(np.sqrt(E))

    @pl.when(t == 0)
    def _gather():
        for i in range(n_super):
            ks_ref[:, i * _BLOCK:(i + 1) * _BLOCK] = \
                k_ref[0, :, i * _SUPER:i * _SUPER + _BLOCK]
            vs_ref[:, i * _BLOCK:(i + 1) * _BLOCK] = \
                v_ref[0, :, i * _SUPER:i * _SUPER + _BLOCK]

    start = pl.multiple_of(jnp.clip(t * _TQ - _ALIGN, 0, S - _LOCW), _ALIGN)
    q = q_ref[0] * temp
    ks = ks_ref[...]
    vs = vs_ref[...]
    kl = k_ref[0, :, pl.ds(start, _LOCW)]
    vl = v_ref[0, :, pl.ds(start, _LOCW)]
    bs = bs_ref[:, pl.ds(t * _TQ, _TQ)]
    bl = bl_ref[:, pl.ds(t * _TQ, _TQ)]

    dk = (((0,), (0,)), ((), ()))
    dv = (((1,), (0,)), ((), ()))
    ss = jax.lax.dot_general(ks, q, dk,
                             preferred_element_type=jnp.float32) + bs
    sl = jax.lax.dot_general(kl, q, dk,
                             preferred_element_type=jnp.float32) + bl

    m = jnp.maximum(jnp.max(ss, axis=0), jnp.max(sl, axis=0))
    ps = jnp.exp(ss - m[None, :])
    plc = jnp.exp(sl - m[None, :])
    denom = jnp.sum(ps, axis=0) + jnp.sum(plc, axis=0)

    out = jax.lax.dot_general(vs, ps, dv, preferred_element_type=jnp.float32)
    out = out + jax.lax.dot_general(vl, plc, dv,
                                    preferred_element_type=jnp.float32)
    o_ref[0] = out / denom[None, :]


def kernel(query, key, value):
    B, T, H, E = query.shape
    S = key.shape[1]
    qt = jnp.transpose(query[0], (1, 2, 0))
    kt = jnp.transpose(key[0], (1, 2, 0))
    vt = jnp.transpose(value[0], (1, 2, 0))
    ns = (S // _SUPER) * _BLOCK
    bias_s, bias_l = _make_biases(T, S)

    out = pl.pallas_call(
        functools.partial(_attn_kernel, H, E),
        grid=(H, T // _TQ),
        in_specs=[
            pl.BlockSpec((1, E, _TQ), lambda h, t: (h, 0, t)),
            pl.BlockSpec((1, E, S), lambda h, t: (h, 0, 0)),
            pl.BlockSpec((1, E, S), lambda h, t: (h, 0, 0)),
            pl.BlockSpec((ns, T), lambda h, t: (0, 0)),
            pl.BlockSpec((_LOCW, T), lambda h, t: (0, 0)),
        ],
        out_specs=pl.BlockSpec((1, E, _TQ), lambda h, t: (h, 0, t)),
        out_shape=jax.ShapeDtypeStruct((H, E, T), jnp.float32),
        scratch_shapes=[
            pltpu.VMEM((E, ns), jnp.float32),
            pltpu.VMEM((E, ns), jnp.float32),
        ],
    )(qt, kt, vt, jnp.asarray(bias_s), jnp.asarray(bias_l))
    return jnp.transpose(out, (2, 0, 1))[None]


# speedup vs baseline: 1.8867x; 1.6413x over previous
import functools

import jax
import jax.numpy as jnp
import numpy as np
from jax.experimental import pallas as pl
from jax.experimental.pallas import tpu as pltpu

_BLOCK = 32
_NLOCAL = 2
_STRIDE = 8
_TQ = 256
_SUPER = _STRIDE * _BLOCK
_LOCW = 448
_ALIGN = 128
_NEG = -1e30


def _local_start(t, S):
    return min(max(t * _TQ - _ALIGN, 0), S - _LOCW)


def _make_biases(T, S):
    ns = (S // _SUPER) * _BLOCK
    rows = np.arange(T)[None, :] // _BLOCK
    cs = np.arange(ns)[:, None] // _BLOCK * _STRIDE
    bias_s = np.where(np.abs(rows - cs) >= _NLOCAL, 0.0, _NEG).astype(np.float32)

    bias_l = np.full((_LOCW, T), _NEG, dtype=np.float32)
    for t in range(T // _TQ):
        start = _local_start(t, S)
        r = np.arange(t * _TQ, (t + 1) * _TQ)[None, :] // _BLOCK
        c = start // _BLOCK + np.arange(_LOCW)[:, None] // _BLOCK
        bias_l[:, t * _TQ:(t + 1) * _TQ] = np.where(
            np.abs(r - c) < _NLOCAL, 0.0, _NEG)
    return bias_s, bias_l


def _attn_kernel(H, E, q_ref, k_ref, v_ref, bs_ref, bl_ref, o_ref,
                 ks_ref, vs_ref):
    S = k_ref.shape[2]
    n_super = S // _SUPER
    temp = 1.0 / float(np.sqrt(E))

    for i in range(n_super):
        ks_ref[:, i * _BLOCK:(i + 1) * _BLOCK] = \
            k_ref[0, :, i * _SUPER:i * _SUPER + _BLOCK]
        vs_ref[:, i * _BLOCK:(i + 1) * _BLOCK] = \
            v_ref[0, :, i * _SUPER:i * _SUPER + _BLOCK]
    ks = ks_ref[...]
    vs = vs_ref[...]

    dk = (((0,), (0,)), ((), ()))
    dv = (((1,), (0,)), ((), ()))
    for t in range(q_ref.shape[2] // _TQ):
        start = _local_start(t, S)
        c0 = t * _TQ
        q = q_ref[0, :, c0:c0 + _TQ] * temp
        kl = k_ref[0, :, start:start + _LOCW]
        vl = v_ref[0, :, start:start + _LOCW]
        bs = bs_ref[:, c0:c0 + _TQ]
        bl = bl_ref[:, c0:c0 + _TQ]

        ss = jax.lax.dot_general(ks, q, dk,
                                 preferred_element_type=jnp.float32) + bs
        sl = jax.lax.dot_general(kl, q, dk,
                                 preferred_element_type=jnp.float32) + bl

        m = jnp.maximum(jnp.max(ss, axis=0), jnp.max(sl, axis=0))
        ps = jnp.exp(ss - m[None, :])
        plc = jnp.exp(sl - m[None, :])
        denom = jnp.sum(ps, axis=0) + jnp.sum(plc, axis=0)

        out = jax.lax.dot_general(vs, ps, dv,
                                  preferred_element_type=jnp.float32)
        out = out + jax.lax.dot_general(vl, plc, dv,
                                        preferred_element_type=jnp.float32)
        o_ref[0, :, c0:c0 + _TQ] = out / denom[None, :]


def kernel(query, key, value):
    B, T, H, E = query.shape
    S = key.shape[1]
    qt = jnp.transpose(query[0], (1, 2, 0))
    kt = jnp.transpose(key[0], (1, 2, 0))
    vt = jnp.transpose(value[0], (1, 2, 0))
    ns = (S // _SUPER) * _BLOCK
    bias_s, bias_l = _make_biases(T, S)

    out = pl.pallas_call(
        functools.partial(_attn_kernel, H, E),
        grid=(H,),
        in_specs=[
            pl.BlockSpec((1, E, T), lambda h: (h, 0, 0)),
            pl.BlockSpec((1, E, S), lambda h: (h, 0, 0)),
            pl.BlockSpec((1, E, S), lambda h: (h, 0, 0)),
            pl.BlockSpec((ns, T), lambda h: (0, 0)),
            pl.BlockSpec((_LOCW, T), lambda h: (0, 0)),
        ],
        out_specs=pl.BlockSpec((1, E, T), lambda h: (h, 0, 0)),
        out_shape=jax.ShapeDtypeStruct((H, E, T), jnp.float32),
        scratch_shapes=[
            pltpu.VMEM((E, ns), jnp.float32),
            pltpu.VMEM((E, ns), jnp.float32),
        ],
    )(qt, kt, vt, jnp.asarray(bias_s), jnp.asarray(bias_l))
    return jnp.transpose(out, (2, 0, 1))[None]


# speedup vs baseline: 2.2211x; 1.1772x over previous
import functools

import jax
import jax.numpy as jnp
import numpy as np
from jax.experimental import pallas as pl
from jax.experimental.pallas import tpu as pltpu

_BLOCK = 32
_NLOCAL = 2
_STRIDE = 8
_TQ = 256
_SUPER = _STRIDE * _BLOCK
_LOCW = _TQ + 2 * _BLOCK
_NEG = -1e30


def _local_start(t, S):
    return min(max(t * _TQ - _BLOCK, 0), S - _LOCW)


def _make_biases(T, S):
    ns = (S // _SUPER) * _BLOCK
    rows = np.arange(T)[None, :] // _BLOCK
    cs = np.arange(ns)[:, None] // _BLOCK * _STRIDE
    bias_s = np.where(np.abs(rows - cs) >= _NLOCAL, 0.0, _NEG).astype(np.float32)

    bias_l = np.full((_LOCW, T), _NEG, dtype=np.float32)
    for t in range(T // _TQ):
        start = _local_start(t, S)
        r = np.arange(t * _TQ, (t + 1) * _TQ)[None, :] // _BLOCK
        c = start // _BLOCK + np.arange(_LOCW)[:, None] // _BLOCK
        bias_l[:, t * _TQ:(t + 1) * _TQ] = np.where(
            np.abs(r - c) < _NLOCAL, 0.0, _NEG)
    return bias_s, bias_l


def _attn_kernel(H, E, q_ref, k_ref, v_ref, bs_ref, bl_ref, o_ref,
                 ks_ref, vs_ref):
    S = k_ref.shape[2]
    n_super = S // _SUPER
    temp = 1.0 / float(np.sqrt(E))

    for i in range(n_super):
        ks_ref[:, i * _BLOCK:(i + 1) * _BLOCK] = \
            k_ref[0, :, i * _SUPER:i * _SUPER + _BLOCK]
        vs_ref[:, i * _BLOCK:(i + 1) * _BLOCK] = \
            v_ref[0, :, i * _SUPER:i * _SUPER + _BLOCK]
    ks = ks_ref[...]
    vs = vs_ref[...]

    dk = (((0,), (0,)), ((), ()))
    dv = (((1,), (0,)), ((), ()))
    for t in range(q_ref.shape[2] // _TQ):
        start = _local_start(t, S)
        c0 = t * _TQ
        q = q_ref[0, :, c0:c0 + _TQ] * temp
        kl = k_ref[0, :, start:start + _LOCW]
        vl = v_ref[0, :, start:start + _LOCW]
        bs = bs_ref[:, c0:c0 + _TQ]
        bl = bl_ref[:, c0:c0 + _TQ]

        ss = jax.lax.dot_general(ks, q, dk,
                                 preferred_element_type=jnp.float32) + bs
        sl = jax.lax.dot_general(kl, q, dk,
                                 preferred_element_type=jnp.float32) + bl

        m = jnp.maximum(jnp.max(ss, axis=0), jnp.max(sl, axis=0))
        ps = jnp.exp(ss - m[None, :])
        plc = jnp.exp(sl - m[None, :])
        denom = jnp.sum(ps, axis=0) + jnp.sum(plc, axis=0)

        out = jax.lax.dot_general(vs, ps, dv,
                                  preferred_element_type=jnp.float32)
        out = out + jax.lax.dot_general(vl, plc, dv,
                                        preferred_element_type=jnp.float32)
        o_ref[0, :, c0:c0 + _TQ] = out / denom[None, :]


def kernel(query, key, value):
    B, T, H, E = query.shape
    S = key.shape[1]
    qt = jnp.transpose(query[0], (1, 2, 0))
    kt = jnp.transpose(key[0], (1, 2, 0))
    vt = jnp.transpose(value[0], (1, 2, 0))
    ns = (S // _SUPER) * _BLOCK
    bias_s, bias_l = _make_biases(T, S)

    out = pl.pallas_call(
        functools.partial(_attn_kernel, H, E),
        grid=(H,),
        in_specs=[
            pl.BlockSpec((1, E, T), lambda h: (h, 0, 0)),
            pl.BlockSpec((1, E, S), lambda h: (h, 0, 0)),
            pl.BlockSpec((1, E, S), lambda h: (h, 0, 0)),
            pl.BlockSpec((ns, T), lambda h: (0, 0)),
            pl.BlockSpec((_LOCW, T), lambda h: (0, 0)),
        ],
        out_specs=pl.BlockSpec((1, E, T), lambda h: (h, 0, 0)),
        out_shape=jax.ShapeDtypeStruct((H, E, T), jnp.float32),
        scratch_shapes=[
            pltpu.VMEM((E, ns), jnp.float32),
            pltpu.VMEM((E, ns), jnp.float32),
        ],
    )(qt, kt, vt, jnp.asarray(bias_s), jnp.asarray(bias_l))
    return jnp.transpose(out, (2, 0, 1))[None]


# speedup vs baseline: 2.2909x; 1.0314x over previous
import functools

import jax
import jax.numpy as jnp
import numpy as np
from jax.experimental import pallas as pl
from jax.experimental.pallas import tpu as pltpu

_BLOCK = 32
_NLOCAL = 2
_STRIDE = 8
_TQ = 256
_SUPER = _STRIDE * _BLOCK
_LOCW = _TQ + 2 * _BLOCK
_NEG = -1e30


def _local_start(t, S):
    return min(max(t * _TQ - _BLOCK, 0), S - _LOCW)


def _make_biases(T, S):
    ns = (S // _SUPER) * _BLOCK
    rows = np.arange(T)[None, :] // _BLOCK
    cs = np.arange(ns)[:, None] // _BLOCK * _STRIDE
    bias_s = np.where(np.abs(rows - cs) >= _NLOCAL, 0.0, _NEG).astype(np.float32)

    bias_l = np.full((_LOCW, T), _NEG, dtype=np.float32)
    for t in range(T // _TQ):
        start = _local_start(t, S)
        r = np.arange(t * _TQ, (t + 1) * _TQ)[None, :] // _BLOCK
        c = start // _BLOCK + np.arange(_LOCW)[:, None] // _BLOCK
        bias_l[:, t * _TQ:(t + 1) * _TQ] = np.where(
            np.abs(r - c) < _NLOCAL, 0.0, _NEG)
    return bias_s, bias_l


def _attn_kernel(H, E, HG, q_ref, k_ref, v_ref, bs_ref, bl_ref, o_ref,
                 ks_ref, vs_ref):
    S = k_ref.shape[2]
    n_super = S // _SUPER
    temp = 1.0 / float(np.sqrt(E))

    dk = (((0,), (0,)), ((), ()))
    dv = (((1,), (0,)), ((), ()))
    for hh in range(HG):
        for i in range(n_super):
            ks_ref[:, i * _BLOCK:(i + 1) * _BLOCK] = \
                k_ref[hh, :, i * _SUPER:i * _SUPER + _BLOCK]
            vs_ref[:, i * _BLOCK:(i + 1) * _BLOCK] = \
                v_ref[hh, :, i * _SUPER:i * _SUPER + _BLOCK]
        ks = ks_ref[...]
        vs = vs_ref[...]

        for t in range(q_ref.shape[2] // _TQ):
            start = _local_start(t, S)
            c0 = t * _TQ
            q = q_ref[hh, :, c0:c0 + _TQ] * temp
            kl = k_ref[hh, :, start:start + _LOCW]
            vl = v_ref[hh, :, start:start + _LOCW]
            bs = bs_ref[:, c0:c0 + _TQ]
            bl = bl_ref[:, c0:c0 + _TQ]

            ss = jax.lax.dot_general(ks, q, dk,
                                     preferred_element_type=jnp.float32) + bs
            sl = jax.lax.dot_general(kl, q, dk,
                                     preferred_element_type=jnp.float32) + bl

            m = jnp.maximum(jnp.max(ss, axis=0), jnp.max(sl, axis=0))
            ps = jnp.exp(ss - m[None, :])
            plc = jnp.exp(sl - m[None, :])
            denom = jnp.sum(ps, axis=0) + jnp.sum(plc, axis=0)

            out = jax.lax.dot_general(vs, ps, dv,
                                      preferred_element_type=jnp.float32)
            out = out + jax.lax.dot_general(vl, plc, dv,
                                            preferred_element_type=jnp.float32)
            o_ref[hh, :, c0:c0 + _TQ] = out / denom[None, :]


def kernel(query, key, value):
    B, T, H, E = query.shape
    S = key.shape[1]
    qt = jnp.transpose(query[0], (1, 2, 0))
    kt = jnp.transpose(key[0], (1, 2, 0))
    vt = jnp.transpose(value[0], (1, 2, 0))
    ns = (S // _SUPER) * _BLOCK
    bias_s, bias_l = _make_biases(T, S)

    HG = 2
    out = pl.pallas_call(
        functools.partial(_attn_kernel, H, E, HG),
        grid=(H // HG,),
        in_specs=[
            pl.BlockSpec((HG, E, T), lambda h: (h, 0, 0)),
            pl.BlockSpec((HG, E, S), lambda h: (h, 0, 0)),
            pl.BlockSpec((HG, E, S), lambda h: (h, 0, 0)),
            pl.BlockSpec((ns, T), lambda h: (0, 0)),
            pl.BlockSpec((_LOCW, T), lambda h: (0, 0)),
        ],
        out_specs=pl.BlockSpec((HG, E, T), lambda h: (h, 0, 0)),
        out_shape=jax.ShapeDtypeStruct((H, E, T), jnp.float32),
        scratch_shapes=[
            pltpu.VMEM((E, ns), jnp.float32),
            pltpu.VMEM((E, ns), jnp.float32),
        ],
    )(qt, kt, vt, jnp.asarray(bias_s), jnp.asarray(bias_l))
    return jnp.transpose(out, (2, 0, 1))[None]
